# baseline (speedup 1.0000x reference)
import functools

import jax
import jax.numpy as jnp
from jax import lax
from jax.experimental import pallas as pl
from jax.experimental.pallas import tpu as pltpu
from jax.experimental.pallas import tpu_sc as plsc

N = 10000
E = 160000
H = 256
HH = 128
L = 6
NSUB = 16
CH = 112
NCHUNK = 90
EPT = NCHUNK * CH
E2 = NSUB * EPT
RPT = 624
TAIL = N - NSUB * RPT
BS = 1000
NBLK = N // BS

_f32 = jnp.float32


def _sc_edge_body(do_cnt, a0_h, a1_h, b0_h, b1_h, src_h, dstg_h, dsts_h,
                  ea_h, w1e_h, *args):
    if do_cnt:
        (s0_h, s1_h, cnt_h, src_c, dg_c, ds_c, ea_c, ab, w1eb,
         ssh, sema, semb, semi, sems) = args
    else:
        (s0_h, s1_h, src_c, dg_c, ds_c, ea_c, ab, w1eb,
         ssh, sema, semb, semi, sems) = args
        cnt_h = None
    c = lax.axis_index("c")
    s = lax.axis_index("s")
    az = ab.at[0]
    oz = ab.at[1]

    def _fill(ref, val):
        v = jnp.full((16,), val, _f32)

        def _fr(r, _):
            row = ref.at[r]
            for j in range(HH // 16):
                row[pl.ds(j * 16, 16)] = v
            return 0

        lax.fori_loop(0, CH, _fr, 0)

    nfull = RPT // CH
    rem = RPT - nfull * CH

    def _zero_ssh():
        base = pl.multiple_of(s * RPT, 8)
        for i in range(nfull):
            pltpu.sync_copy(az, ssh.at[pl.ds(base + i * CH, CH)])
        pltpu.sync_copy(az.at[pl.ds(0, rem)],
                        ssh.at[pl.ds(base + nfull * CH, rem)])

        @pl.when(s == NSUB - 1)
        def _():
            pltpu.sync_copy(az.at[pl.ds(0, TAIL)],
                            ssh.at[pl.ds(NSUB * RPT, TAIL)])

    def _write_ssh(dst_hbm):
        base = pl.multiple_of(s * RPT, 8)
        pltpu.sync_copy(ssh.at[pl.ds(base, RPT)],
                        dst_hbm.at[pl.ds(base, RPT)])

        @pl.when(s == NSUB - 1)
        def _():
            pltpu.sync_copy(ssh.at[pl.ds(NSUB * RPT, TAIL)],
                            dst_hbm.at[pl.ds(NSUB * RPT, TAIL)])

    _fill(az, 0.0)
    _zero_ssh()
    plsc.subcore_barrier()

    def _idx_issue(kk, sl):
        off = pl.multiple_of(s * EPT + kk * CH, 8)
        pltpu.async_copy(src_h.at[pl.ds(off, CH)], src_c.at[sl], semi)
        pltpu.async_copy(dstg_h.at[pl.ds(off, CH)], dg_c.at[sl], semi)
        pltpu.async_copy(dsts_h.at[pl.ds(off, CH)], ds_c.at[sl], semi)
        pltpu.async_copy(ea_h.at[pl.ds(off, CH)], ea_c.at[sl], semi)

    def _idx_wait(kk, sl):
        off = pl.multiple_of(s * EPT + kk * CH, 8)
        pltpu.make_async_copy(src_h.at[pl.ds(off, CH)], src_c.at[sl],
                              semi).wait()
        pltpu.make_async_copy(dstg_h.at[pl.ds(off, CH)], dg_c.at[sl],
                              semi).wait()
        pltpu.make_async_copy(dsts_h.at[pl.ds(off, CH)], ds_c.at[sl],
                              semi).wait()
        pltpu.make_async_copy(ea_h.at[pl.ds(off, CH)], ea_c.at[sl],
                              semi).wait()

    def _run_half(A_h, B_h, S_h, hi):
        pltpu.sync_copy(w1e_h.at[hi], w1eb)
        w1row = w1eb.at[0]
        w1 = [w1row[pl.ds(j * 16, 16)] for j in range(HH // 16)]

        off0 = pl.multiple_of(s * EPT, 8)
        pltpu.sync_copy(src_h.at[pl.ds(off0, CH)], src_c.at[0])
        pltpu.sync_copy(dstg_h.at[pl.ds(off0, CH)], dg_c.at[0])
        pltpu.sync_copy(dsts_h.at[pl.ds(off0, CH)], ds_c.at[0])
        pltpu.sync_copy(ea_h.at[pl.ds(off0, CH)], ea_c.at[0])
        off1 = pl.multiple_of(s * EPT + CH, 8)
        pltpu.sync_copy(src_h.at[pl.ds(off1, CH)], src_c.at[1])
        pltpu.sync_copy(dstg_h.at[pl.ds(off1, CH)], dg_c.at[1])
        pltpu.sync_copy(dsts_h.at[pl.ds(off1, CH)], ds_c.at[1])
        pltpu.sync_copy(ea_h.at[pl.ds(off1, CH)], ea_c.at[1])
        _idx_issue(2, 2)
        pltpu.async_copy(A_h.at[dg_c.at[0]], ab.at[0], sema).wait()
        pltpu.async_copy(B_h.at[src_c.at[0]], ab.at[0], semb,
                         add=True).wait()
        pltpu.async_copy(A_h.at[dg_c.at[1]], ab.at[1], sema)

        def _chunk(k, _):
            p3 = lax.rem(k, 3)
            s1 = lax.rem(k + 1, 3)
            s2 = lax.rem(k + 2, 3)
            j2 = lax.rem(k + 2, 4)
            j3 = lax.rem(k + 3, 4)
            kn1 = jnp.minimum(k + 1, NCHUNK - 1)
            kn2 = jnp.minimum(k + 2, NCHUNK - 1)
            kn3 = jnp.minimum(k + 3, NCHUNK - 1)
            _idx_wait(kn2, j2)

            @pl.when(k > 0)
            def _():
                pltpu.make_async_copy(ab.at[s2], ssh.at[ds_c.at[0]],
                                      sems).wait()

            pltpu.make_async_copy(A_h.at[dg_c.at[0]], ab.at[s1],
                                  sema).wait()
            pltpu.async_copy(B_h.at[src_c.at[lax.rem(kn1, 4)]], ab.at[s1],
                             semb, add=True)
            pltpu.async_copy(A_h.at[dg_c.at[j2]], ab.at[s2], sema)
            _idx_issue(kn3, j3)
            az_ = ab.at[p3]
            ek = ea_c.at[lax.rem(k, 4)]

            def _grp(g, _):
                ea16 = ek[pl.ds(g * 16, 16)]

                def _lane(h2, _):
                    for i in range(2):
                        l = h2 * 2 + i
                        r = g * 16 + l
                        eav = ea16[jnp.full((16,), l, jnp.int32)]
                        arow = az_.at[r]
                        for j in range(HH // 16):
                            aj = arow[pl.ds(j * 16, 16)]
                            arow[pl.ds(j * 16, 16)] = jnp.maximum(
                                aj + eav * w1[j], 0.0)
                    return 0

                lax.fori_loop(0, 8, _lane, 0)
                return 0

            lax.fori_loop(0, CH // 16, _grp, 0)
            pltpu.async_copy(az_, ssh.at[ds_c.at[lax.rem(k, 4)]], sems,
                             add=True)
            pltpu.make_async_copy(B_h.at[src_c.at[0]], ab.at[s1],
                                  semb).wait()
            return 0

        lax.fori_loop(0, NCHUNK, _chunk, 0)
        pltpu.make_async_copy(ab.at[0], ssh.at[ds_c.at[0]], sems).wait()
        pltpu.make_async_copy(A_h.at[dg_c.at[0]], ab.at[0], sema).wait()
        _idx_wait(0, 0)
        plsc.subcore_barrier()
        _write_ssh(S_h)

    @pl.when(c == 0)
    def _():
        _run_half(a0_h, b0_h, s0_h, 0)
        if do_cnt:
            plsc.subcore_barrier()
            _fill(az, 0.0)
            _zero_ssh()
            _fill(oz, 1.0)
            plsc.subcore_barrier()

            def _cchunk(k, _):
                off = pl.multiple_of(s * EPT + k * CH, 8)
                dk = ds_c.at[0]
                pltpu.sync_copy(dsts_h.at[pl.ds(off, CH)], dk)
                pltpu.sync_copy(oz, ssh.at[dk], add=True)
                return 0

            lax.fori_loop(0, NCHUNK, _cchunk, 0)
            plsc.subcore_barrier()
            _write_ssh(cnt_h)

    @pl.when(c == 1)
    def _():
        _run_half(a1_h, b1_h, s1_h, 1)


def _make_sc_edge(do_cnt):
    out_type = [jax.ShapeDtypeStruct((N, HH), _f32),
                jax.ShapeDtypeStruct((N, HH), _f32)]
    if do_cnt:
        out_type = out_type + [jax.ShapeDtypeStruct((N, HH), _f32)]
    scratch = [pltpu.VMEM((4, CH), jnp.int32),
               pltpu.VMEM((4, CH), jnp.int32),
               pltpu.VMEM((4, CH), jnp.int32),
               pltpu.VMEM((4, CH), _f32),
               pltpu.VMEM((3, CH, HH), _f32),
               pltpu.VMEM((1, HH), _f32)]
    scratch = scratch + [
               pltpu.VMEM_SHARED((N + 8, HH), _f32),
               pltpu.SemaphoreType.DMA,
               pltpu.SemaphoreType.DMA,
               pltpu.SemaphoreType.DMA,
               pltpu.SemaphoreType.DMA]
    mesh = plsc.VectorSubcoreMesh(core_axis_name="c", subcore_axis_name="s")
    return functools.partial(
        pl.kernel, mesh=mesh, out_type=out_type, scratch_types=scratch,
    )(functools.partial(_sc_edge_body, do_cnt))


_SC_CACHE = {}


def _sc_edge_first(*args):
    if True not in _SC_CACHE:
        _SC_CACHE[True] = _make_sc_edge(True)
    return _SC_CACHE[True](*args)


def _sc_edge(*args):
    if False not in _SC_CACHE:
        _SC_CACHE[False] = _make_sc_edge(False)
    return _SC_CACHE[False](*args)


def _tc_pre_body(xp, nwp, nbr, cnd, cw1p, cb1r, cw2, cb2r, tp, tw1p, tb1r,
                 tw2, tb2r, w1a, w1b, b1r, h_out, a0, a1, b0, b1o):
    cond = jnp.dot(jax.nn.relu(
        jnp.dot(cnd[...], cw1p[...], preferred_element_type=_f32) + cb1r[...]),
        cw2[...], preferred_element_type=_f32) + cb2r[...]
    tf = jnp.dot(jax.nn.relu(
        jnp.dot(tp[...], tw1p[...], preferred_element_type=_f32) + tb1r[...]),
        tw2[...], preferred_element_type=_f32) + tb2r[...]
    h = (jnp.dot(xp[...], nwp[...], preferred_element_type=_f32)
         + nbr[...] + cond + tf)
    h_out[...] = h
    a = jnp.dot(h, w1a[...], preferred_element_type=_f32) + b1r[...]
    b = jnp.dot(h, w1b[...], preferred_element_type=_f32)
    a0[...] = a[:, :HH]
    a1[...] = a[:, HH:]
    b0[...] = b[:, :HH]
    b1o[...] = b[:, HH:]


def _agg_h(s0, s1, cnt, h, w2, b2r):
    cntv = cnt[...][:, :1]
    inv = 1.0 / jnp.maximum(cntv, 1.0)
    has = (cntv > 0.0).astype(_f32)
    w2v = w2[...]
    agg = (jnp.dot(s0[...], w2v[:HH, :], preferred_element_type=_f32)
           + jnp.dot(s1[...], w2v[HH:, :], preferred_element_type=_f32))
    agg = inv * agg + has * b2r[...]
    return jnp.maximum(agg, 0.0) + h[...]


def _tc_mid_body(s0, s1, cnt, h, w2, b2r, w1a, w1b, b1r,
                 h_out, a0, a1, b0, b1o):
    hn = _agg_h(s0, s1, cnt, h, w2, b2r)
    h_out[...] = hn
    a = jnp.dot(hn, w1a[...], preferred_element_type=_f32) + b1r[...]
    b = jnp.dot(hn, w1b[...], preferred_element_type=_f32)
    a0[...] = a[:, :HH]
    a1[...] = a[:, HH:]
    b0[...] = b[:, :HH]
    b1o[...] = b[:, HH:]


def _tc_last_body(s0, s1, cnt, h, w2, b2r, ow1, ob1r, ow2p, ob2r, out):
    hn = _agg_h(s0, s1, cnt, h, w2, b2r)
    o = jnp.dot(jax.nn.relu(
        jnp.dot(hn, ow1[...], preferred_element_type=_f32) + ob1r[...]),
        ow2p[...], preferred_element_type=_f32) + ob2r[...]
    out[...] = o


def _row_spec(w):
    return pl.BlockSpec((BS, w), lambda i: (i, 0))


def _full_spec(shape):
    nd = len(shape)
    return pl.BlockSpec(shape, lambda i: (0,) * nd)


_W = _full_spec((H, H))
_R = _full_spec((1, H))

_tc_pre = pl.pallas_call(
    _tc_pre_body,
    grid=(NBLK,),
    in_specs=[_row_spec(HH), _full_spec((HH, H)), _R,
              _full_spec((1, HH)), _full_spec((HH, H)), _R, _W, _R,
              _full_spec((1, HH)), _full_spec((HH, H)), _R, _W, _R,
              _W, _W, _R],
    out_specs=[_row_spec(H), _row_spec(HH), _row_spec(HH),
               _row_spec(HH), _row_spec(HH)],
    out_shape=[jax.ShapeDtypeStruct((N, H), _f32)] +
              [jax.ShapeDtypeStruct((N, HH), _f32)] * 4,
)

_tc_mid = pl.pallas_call(
    _tc_mid_body,
    grid=(NBLK,),
    in_specs=[_row_spec(HH), _row_spec(HH), _row_spec(HH), _row_spec(H),
              _W, _R, _W, _W, _R],
    out_specs=[_row_spec(H), _row_spec(HH), _row_spec(HH),
               _row_spec(HH), _row_spec(HH)],
    out_shape=[jax.ShapeDtypeStruct((N, H), _f32)] +
              [jax.ShapeDtypeStruct((N, HH), _f32)] * 4,
)

_tc_last = pl.pallas_call(
    _tc_last_body,
    grid=(NBLK,),
    in_specs=[_row_spec(HH), _row_spec(HH), _row_spec(HH), _row_spec(H),
              _W, _R, _W, _R, _full_spec((H, HH)), _full_spec((1, HH))],
    out_specs=_row_spec(HH),
    out_shape=jax.ShapeDtypeStruct((N, HH), _f32),
)


def kernel(x, edge_index, edge_attr, t, condition, cW1, cb1, cW2, cb2, nW, nb,
           tW1, tb1, tW2, tb2, convW1, convb1, convW2, convb2, oW1, ob1, oW2,
           ob2):
    ppt = EPT - E // NSUB
    zpad = jnp.zeros((NSUB, ppt), jnp.int32)
    tpad = jnp.broadcast_to(
        N + (jnp.arange(ppt, dtype=jnp.int32) % 8)[None, :], (NSUB, ppt))
    src2 = jnp.concatenate(
        [edge_index[0].reshape(NSUB, -1), zpad], axis=1).reshape(E2)
    dstg = jnp.concatenate(
        [edge_index[1].reshape(NSUB, -1), zpad], axis=1).reshape(E2)
    dsts = jnp.concatenate(
        [edge_index[1].reshape(NSUB, -1), tpad], axis=1).reshape(E2)
    ea2 = jnp.concatenate(
        [edge_attr.reshape(NSUB, -1), zpad.astype(_f32)],
        axis=1).reshape(E2)

    xp = jnp.pad(x, ((0, 0), (0, HH - x.shape[1])))
    nwp = jnp.pad(nW, ((0, HH - nW.shape[0]), (0, 0)))
    cnd = jnp.pad(condition.reshape(1, -1), ((0, 0), (0, HH - condition.shape[0])))
    cw1p = jnp.pad(cW1, ((0, HH - cW1.shape[0]), (0, 0)))
    tp = jnp.pad(t.reshape(1, 1), ((0, 0), (0, HH - 1)))
    tw1p = jnp.pad(tW1, ((0, HH - tW1.shape[0]), (0, 0)))
    ow2p = jnp.pad(oW2, ((0, 0), (0, HH - oW2.shape[1])))
    ob2r = jnp.pad(ob2.reshape(1, -1), ((0, 0), (0, HH - ob2.shape[0])))
    w1e = convW1[:, 2 * H, :].reshape(L, 2, 1, HH)

    h, a0, a1, b0, b1 = _tc_pre(
        xp, nwp, nb.reshape(1, H), cnd, cw1p, cb1.reshape(1, H), cW2,
        cb2.reshape(1, H), tp, tw1p, tb1.reshape(1, H), tW2,
        tb2.reshape(1, H), convW1[0][:H], convW1[0][H:2 * H],
        convb1[0].reshape(1, H))

    s0, s1, cnt = _sc_edge_first(a0, a1, b0, b1, src2, dstg, dsts, ea2,
                                 w1e[0])

    for i in range(1, L):
        h, a0, a1, b0, b1 = _tc_mid(
            s0, s1, cnt, h, convW2[i - 1], convb2[i - 1].reshape(1, H),
            convW1[i][:H], convW1[i][H:2 * H], convb1[i].reshape(1, H))
        s0, s1 = _sc_edge(a0, a1, b0, b1, src2, dstg, dsts, ea2, w1e[i])

    out = _tc_last(s0, s1, cnt, h, convW2[L - 1], convb2[L - 1].reshape(1, H),
                   oW1, ob1.reshape(1, H), ow2p, ob2r)
    return out[:, :x.shape[1]]


# speedup vs baseline: 1.4700x; 1.4700x over previous
import functools

import jax
import jax.numpy as jnp
from jax import lax
from jax.experimental import pallas as pl
from jax.experimental.pallas import tpu as pltpu
from jax.experimental.pallas import tpu_sc as plsc

N = 10000
E = 160000
H = 256
HH = 128
L = 6
NSUB = 16
EPT = E // NSUB
CH = 80
NCHUNK = EPT // CH
RPT = 624
TAIL = N - NSUB * RPT
BS = 1000
NBLK = N // BS

_f32 = jnp.float32


def _sc_edge_body(do_cnt, a0_h, a1_h, b0_h, b1_h, src_h, dst_h, ea_h,
                  w1e_h, *args):
    if do_cnt:
        (s0_h, s1_h, cnt_h, src_c, dst_c, ea_c, ab, w1eb,
         ssh, sema, semb, semi, sems) = args
    else:
        (s0_h, s1_h, src_c, dst_c, ea_c, ab, w1eb,
         ssh, sema, semb, semi, sems) = args
        cnt_h = None
    c = lax.axis_index("c")
    s = lax.axis_index("s")
    az = ab.at[0]
    oz = ab.at[1]

    def _fill(ref, val):
        v = jnp.full((16,), val, _f32)

        def _fr(r, _):
            row = ref.at[r]
            for j in range(HH // 16):
                row[pl.ds(j * 16, 16)] = v
            return 0

        lax.fori_loop(0, CH, _fr, 0)

    nfull = RPT // CH
    rem = RPT - nfull * CH

    def _zero_ssh():
        base = pl.multiple_of(s * RPT, 8)
        for i in range(nfull):
            pltpu.sync_copy(az, ssh.at[pl.ds(base + i * CH, CH)])
        pltpu.sync_copy(az.at[pl.ds(0, rem)],
                        ssh.at[pl.ds(base + nfull * CH, rem)])

        @pl.when(s == NSUB - 1)
        def _():
            pltpu.sync_copy(az.at[pl.ds(0, TAIL)],
                            ssh.at[pl.ds(NSUB * RPT, TAIL)])

    def _write_ssh(dst_hbm):
        base = pl.multiple_of(s * RPT, 8)
        pltpu.sync_copy(ssh.at[pl.ds(base, RPT)],
                        dst_hbm.at[pl.ds(base, RPT)])

        @pl.when(s == NSUB - 1)
        def _():
            pltpu.sync_copy(ssh.at[pl.ds(NSUB * RPT, TAIL)],
                            dst_hbm.at[pl.ds(NSUB * RPT, TAIL)])

    _fill(az, 0.0)
    _zero_ssh()
    plsc.subcore_barrier()

    def _idx_issue(kk, sl):
        off = pl.multiple_of(s * EPT + kk * CH, 8)
        pltpu.async_copy(src_h.at[pl.ds(off, CH)], src_c.at[sl], semi)
        pltpu.async_copy(dst_h.at[pl.ds(off, CH)], dst_c.at[sl], semi)
        pltpu.async_copy(ea_h.at[pl.ds(off, CH)], ea_c.at[sl], semi)

    def _idx_wait(kk, sl):
        off = pl.multiple_of(s * EPT + kk * CH, 8)
        pltpu.make_async_copy(src_h.at[pl.ds(off, CH)], src_c.at[sl],
                              semi).wait()
        pltpu.make_async_copy(dst_h.at[pl.ds(off, CH)], dst_c.at[sl],
                              semi).wait()
        pltpu.make_async_copy(ea_h.at[pl.ds(off, CH)], ea_c.at[sl],
                              semi).wait()

    def _run_half(A_h, B_h, S_h, hi):
        pltpu.sync_copy(w1e_h.at[hi], w1eb)
        w1row = w1eb.at[0]
        w1 = [w1row[pl.ds(j * 16, 16)] for j in range(HH // 16)]

        off0 = pl.multiple_of(s * EPT, 8)
        pltpu.sync_copy(src_h.at[pl.ds(off0, CH)], src_c.at[0])
        pltpu.sync_copy(dst_h.at[pl.ds(off0, CH)], dst_c.at[0])
        pltpu.sync_copy(ea_h.at[pl.ds(off0, CH)], ea_c.at[0])
        off1 = pl.multiple_of(s * EPT + CH, 8)
        pltpu.sync_copy(src_h.at[pl.ds(off1, CH)], src_c.at[1])
        pltpu.sync_copy(dst_h.at[pl.ds(off1, CH)], dst_c.at[1])
        pltpu.sync_copy(ea_h.at[pl.ds(off1, CH)], ea_c.at[1])
        _idx_issue(2, 2)
        pltpu.async_copy(A_h.at[dst_c.at[0]], ab.at[0], sema).wait()
        pltpu.async_copy(B_h.at[src_c.at[0]], ab.at[0], semb,
                         add=True).wait()
        pltpu.async_copy(A_h.at[dst_c.at[1]], ab.at[1], sema)

        def _chunk(k, _):
            p3 = lax.rem(k, 3)
            s1 = lax.rem(k + 1, 3)
            s2 = lax.rem(k + 2, 3)
            j2 = lax.rem(k + 2, 4)
            j3 = lax.rem(k + 3, 4)
            kn1 = jnp.minimum(k + 1, NCHUNK - 1)
            kn2 = jnp.minimum(k + 2, NCHUNK - 1)
            kn3 = jnp.minimum(k + 3, NCHUNK - 1)
            _idx_wait(kn2, j2)

            @pl.when(k > 0)
            def _():
                pltpu.make_async_copy(ab.at[s2], ssh.at[dst_c.at[0]],
                                      sems).wait()

            pltpu.make_async_copy(A_h.at[dst_c.at[0]], ab.at[s1],
                                  sema).wait()
            pltpu.async_copy(B_h.at[src_c.at[lax.rem(kn1, 4)]], ab.at[s1],
                             semb, add=True)
            pltpu.async_copy(A_h.at[dst_c.at[j2]], ab.at[s2], sema)
            _idx_issue(kn3, j3)
            az_ = ab.at[p3]
            ek = ea_c.at[lax.rem(k, 4)]

            def _grp(g, _):
                ea16 = ek[pl.ds(g * 16, 16)]

                def _lane(h2, _):
                    for i in range(2):
                        l = h2 * 2 + i
                        r = g * 16 + l
                        eav = ea16[jnp.full((16,), l, jnp.int32)]
                        arow = az_.at[r]
                        for j in range(HH // 16):
                            aj = arow[pl.ds(j * 16, 16)]
                            arow[pl.ds(j * 16, 16)] = jnp.maximum(
                                aj + eav * w1[j], 0.0)
                    return 0

                lax.fori_loop(0, 8, _lane, 0)
                return 0

            lax.fori_loop(0, CH // 16, _grp, 0)
            pltpu.async_copy(az_, ssh.at[dst_c.at[lax.rem(k, 4)]], sems,
                             add=True)
            pltpu.make_async_copy(B_h.at[src_c.at[0]], ab.at[s1],
                                  semb).wait()
            return 0

        lax.fori_loop(0, NCHUNK, _chunk, 0)
        pltpu.make_async_copy(ab.at[0], ssh.at[dst_c.at[0]], sems).wait()
        pltpu.make_async_copy(A_h.at[dst_c.at[0]], ab.at[0], sema).wait()
        _idx_wait(0, 0)
        plsc.subcore_barrier()
        _write_ssh(S_h)

    @pl.when(c == 0)
    def _():
        _run_half(a0_h, b0_h, s0_h, 0)
        if do_cnt:
            plsc.subcore_barrier()
            _fill(az, 0.0)
            _zero_ssh()
            _fill(oz, 1.0)
            plsc.subcore_barrier()

            def _cchunk(k, _):
                off = pl.multiple_of(s * EPT + k * CH, 8)
                dk = dst_c.at[0]
                pltpu.sync_copy(dst_h.at[pl.ds(off, CH)], dk)
                pltpu.sync_copy(oz, ssh.at[dk], add=True)
                return 0

            lax.fori_loop(0, NCHUNK, _cchunk, 0)
            plsc.subcore_barrier()
            _write_ssh(cnt_h)

    @pl.when(c == 1)
    def _():
        _run_half(a1_h, b1_h, s1_h, 1)


def _make_sc_edge(do_cnt):
    out_type = [jax.ShapeDtypeStruct((N, HH), _f32),
                jax.ShapeDtypeStruct((N, HH), _f32)]
    if do_cnt:
        out_type = out_type + [jax.ShapeDtypeStruct((N, HH), _f32)]
    scratch = [pltpu.VMEM((4, CH), jnp.int32),
               pltpu.VMEM((4, CH), jnp.int32),
               pltpu.VMEM((4, CH), _f32),
               pltpu.VMEM((3, CH, HH), _f32),
               pltpu.VMEM((1, HH), _f32)]
    scratch = scratch + [
               pltpu.VMEM_SHARED((N, HH), _f32),
               pltpu.SemaphoreType.DMA,
               pltpu.SemaphoreType.DMA,
               pltpu.SemaphoreType.DMA,
               pltpu.SemaphoreType.DMA]
    mesh = plsc.VectorSubcoreMesh(core_axis_name="c", subcore_axis_name="s")
    return functools.partial(
        pl.kernel, mesh=mesh, out_type=out_type, scratch_types=scratch,
    )(functools.partial(_sc_edge_body, do_cnt))


_SC_CACHE = {}


def _sc_edge_first(*args):
    if True not in _SC_CACHE:
        _SC_CACHE[True] = _make_sc_edge(True)
    return _SC_CACHE[True](*args)


def _sc_edge(*args):
    if False not in _SC_CACHE:
        _SC_CACHE[False] = _make_sc_edge(False)
    return _SC_CACHE[False](*args)


def _tc_pre_body(xp, nwp, nbr, cnd, cw1p, cb1r, cw2, cb2r, tp, tw1p, tb1r,
                 tw2, tb2r, w1a, w1b, b1r, h_out, a0, a1, b0, b1o):
    cond = jnp.dot(jax.nn.relu(
        jnp.dot(cnd[...], cw1p[...], preferred_element_type=_f32) + cb1r[...]),
        cw2[...], preferred_element_type=_f32) + cb2r[...]
    tf = jnp.dot(jax.nn.relu(
        jnp.dot(tp[...], tw1p[...], preferred_element_type=_f32) + tb1r[...]),
        tw2[...], preferred_element_type=_f32) + tb2r[...]
    h = (jnp.dot(xp[...], nwp[...], preferred_element_type=_f32)
         + nbr[...] + cond + tf)
    h_out[...] = h
    a = jnp.dot(h, w1a[...], preferred_element_type=_f32) + b1r[...]
    b = jnp.dot(h, w1b[...], preferred_element_type=_f32)
    a0[...] = a[:, :HH]
    a1[...] = a[:, HH:]
    b0[...] = b[:, :HH]
    b1o[...] = b[:, HH:]


def _agg_h(s0, s1, cnt, h, w2, b2r):
    cntv = cnt[...][:, :1]
    inv = 1.0 / jnp.maximum(cntv, 1.0)
    has = (cntv > 0.0).astype(_f32)
    w2v = w2[...]
    agg = (jnp.dot(s0[...], w2v[:HH, :], preferred_element_type=_f32)
           + jnp.dot(s1[...], w2v[HH:, :], preferred_element_type=_f32))
    agg = inv * agg + has * b2r[...]
    return jnp.maximum(agg, 0.0) + h[...]


def _tc_mid_body(s0, s1, cnt, h, w2, b2r, w1a, w1b, b1r,
                 h_out, a0, a1, b0, b1o):
    hn = _agg_h(s0, s1, cnt, h, w2, b2r)
    h_out[...] = hn
    a = jnp.dot(hn, w1a[...], preferred_element_type=_f32) + b1r[...]
    b = jnp.dot(hn, w1b[...], preferred_element_type=_f32)
    a0[...] = a[:, :HH]
    a1[...] = a[:, HH:]
    b0[...] = b[:, :HH]
    b1o[...] = b[:, HH:]


def _tc_last_body(s0, s1, cnt, h, w2, b2r, ow1, ob1r, ow2p, ob2r, out):
    hn = _agg_h(s0, s1, cnt, h, w2, b2r)
    o = jnp.dot(jax.nn.relu(
        jnp.dot(hn, ow1[...], preferred_element_type=_f32) + ob1r[...]),
        ow2p[...], preferred_element_type=_f32) + ob2r[...]
    out[...] = o


def _row_spec(w):
    return pl.BlockSpec((BS, w), lambda i: (i, 0))


def _full_spec(shape):
    nd = len(shape)
    return pl.BlockSpec(shape, lambda i: (0,) * nd)


_W = _full_spec((H, H))
_R = _full_spec((1, H))

_tc_pre = pl.pallas_call(
    _tc_pre_body,
    grid=(NBLK,),
    in_specs=[_row_spec(HH), _full_spec((HH, H)), _R,
              _full_spec((1, HH)), _full_spec((HH, H)), _R, _W, _R,
              _full_spec((1, HH)), _full_spec((HH, H)), _R, _W, _R,
              _W, _W, _R],
    out_specs=[_row_spec(H), _row_spec(HH), _row_spec(HH),
               _row_spec(HH), _row_spec(HH)],
    out_shape=[jax.ShapeDtypeStruct((N, H), _f32)] +
              [jax.ShapeDtypeStruct((N, HH), _f32)] * 4,
)

_tc_mid = pl.pallas_call(
    _tc_mid_body,
    grid=(NBLK,),
    in_specs=[_row_spec(HH), _row_spec(HH), _row_spec(HH), _row_spec(H),
              _W, _R, _W, _W, _R],
    out_specs=[_row_spec(H), _row_spec(HH), _row_spec(HH),
               _row_spec(HH), _row_spec(HH)],
    out_shape=[jax.ShapeDtypeStruct((N, H), _f32)] +
              [jax.ShapeDtypeStruct((N, HH), _f32)] * 4,
)

_tc_last = pl.pallas_call(
    _tc_last_body,
    grid=(NBLK,),
    in_specs=[_row_spec(HH), _row_spec(HH), _row_spec(HH), _row_spec(H),
              _W, _R, _W, _R, _full_spec((H, HH)), _full_spec((1, HH))],
    out_specs=_row_spec(HH),
    out_shape=jax.ShapeDtypeStruct((N, HH), _f32),
)


def kernel(x, edge_index, edge_attr, t, condition, cW1, cb1, cW2, cb2, nW, nb,
           tW1, tb1, tW2, tb2, convW1, convb1, convW2, convb2, oW1, ob1, oW2,
           ob2):
    src2 = edge_index[0]
    dst2 = edge_index[1]
    ea2 = edge_attr.reshape(E)

    xp = jnp.pad(x, ((0, 0), (0, HH - x.shape[1])))
    nwp = jnp.pad(nW, ((0, HH - nW.shape[0]), (0, 0)))
    cnd = jnp.pad(condition.reshape(1, -1), ((0, 0), (0, HH - condition.shape[0])))
    cw1p = jnp.pad(cW1, ((0, HH - cW1.shape[0]), (0, 0)))
    tp = jnp.pad(t.reshape(1, 1), ((0, 0), (0, HH - 1)))
    tw1p = jnp.pad(tW1, ((0, HH - tW1.shape[0]), (0, 0)))
    ow2p = jnp.pad(oW2, ((0, 0), (0, HH - oW2.shape[1])))
    ob2r = jnp.pad(ob2.reshape(1, -1), ((0, 0), (0, HH - ob2.shape[0])))
    w1e = convW1[:, 2 * H, :].reshape(L, 2, 1, HH)

    h, a0, a1, b0, b1 = _tc_pre(
        xp, nwp, nb.reshape(1, H), cnd, cw1p, cb1.reshape(1, H), cW2,
        cb2.reshape(1, H), tp, tw1p, tb1.reshape(1, H), tW2,
        tb2.reshape(1, H), convW1[0][:H], convW1[0][H:2 * H],
        convb1[0].reshape(1, H))

    s0, s1, cnt = _sc_edge_first(a0, a1, b0, b1, src2, dst2, ea2, w1e[0])

    for i in range(1, L):
        h, a0, a1, b0, b1 = _tc_mid(
            s0, s1, cnt, h, convW2[i - 1], convb2[i - 1].reshape(1, H),
            convW1[i][:H], convW1[i][H:2 * H], convb1[i].reshape(1, H))
        s0, s1 = _sc_edge(a0, a1, b0, b1, src2, dst2, ea2, w1e[i])

    out = _tc_last(s0, s1, cnt, h, convW2[L - 1], convb2[L - 1].reshape(1, H),
                   oW1, ob1.reshape(1, H), ow2p, ob2r)
    return out[:, :x.shape[1]]


# speedup vs baseline: 1.5116x; 1.0283x over previous
import functools

import jax
import jax.numpy as jnp
from jax import lax
from jax.experimental import pallas as pl
from jax.experimental.pallas import tpu as pltpu
from jax.experimental.pallas import tpu_sc as plsc

N = 10000
E = 160000
H = 256
HH = 128
L = 6
NSUB = 16
EPT = E // NSUB
CH = 80
NCHUNK = EPT // CH
RPT = 624
TAIL = N - NSUB * RPT
BS = 1000
NBLK = N // BS

_f32 = jnp.float32


def _sc_edge_body(do_cnt, a0_h, a1_h, b0_h, b1_h, src_h, dst_h, ea_h,
                  w1e_h, *args):
    if do_cnt:
        (s0_h, s1_h, cnt_h, cnt1_h, src_c, dst_c, ea_c, ab, w1eb,
         ssh, sema, semb, semi, sems) = args
    else:
        (s0_h, s1_h, src_c, dst_c, ea_c, ab, w1eb,
         ssh, sema, semb, semi, sems) = args
        cnt_h = cnt1_h = None
    c = lax.axis_index("c")
    s = lax.axis_index("s")
    az = ab.at[0]
    oz = ab.at[1]

    def _fill(ref, val):
        v = jnp.full((16,), val, _f32)

        def _fr(r, _):
            row = ref.at[r]
            for j in range(HH // 16):
                row[pl.ds(j * 16, 16)] = v
            return 0

        lax.fori_loop(0, CH, _fr, 0)

    nfull = RPT // CH
    rem = RPT - nfull * CH

    def _zero_ssh():
        base = pl.multiple_of(s * RPT, 8)
        for i in range(nfull):
            pltpu.sync_copy(az, ssh.at[pl.ds(base + i * CH, CH)])
        pltpu.sync_copy(az.at[pl.ds(0, rem)],
                        ssh.at[pl.ds(base + nfull * CH, rem)])

        @pl.when(s == NSUB - 1)
        def _():
            pltpu.sync_copy(az.at[pl.ds(0, TAIL)],
                            ssh.at[pl.ds(NSUB * RPT, TAIL)])

    def _write_ssh(dst_hbm):
        base = pl.multiple_of(s * RPT, 8)
        pltpu.sync_copy(ssh.at[pl.ds(base, RPT)],
                        dst_hbm.at[pl.ds(base, RPT)])

        @pl.when(s == NSUB - 1)
        def _():
            pltpu.sync_copy(ssh.at[pl.ds(NSUB * RPT, TAIL)],
                            dst_hbm.at[pl.ds(NSUB * RPT, TAIL)])

    _fill(az, 0.0)
    _zero_ssh()
    plsc.subcore_barrier()

    def _idx_issue(kk, sl):
        off = pl.multiple_of(s * EPT + kk * CH, 8)
        pltpu.async_copy(src_h.at[pl.ds(off, CH)], src_c.at[sl], semi)
        pltpu.async_copy(dst_h.at[pl.ds(off, CH)], dst_c.at[sl], semi)
        pltpu.async_copy(ea_h.at[pl.ds(off, CH)], ea_c.at[sl], semi)

    def _idx_wait(kk, sl):
        off = pl.multiple_of(s * EPT + kk * CH, 8)
        pltpu.make_async_copy(src_h.at[pl.ds(off, CH)], src_c.at[sl],
                              semi).wait()
        pltpu.make_async_copy(dst_h.at[pl.ds(off, CH)], dst_c.at[sl],
                              semi).wait()
        pltpu.make_async_copy(ea_h.at[pl.ds(off, CH)], ea_c.at[sl],
                              semi).wait()

    def _run_half(A_h, B_h, S_h, hi):
        pltpu.sync_copy(w1e_h.at[hi], w1eb)
        w1row = w1eb.at[0]
        w1 = [w1row[pl.ds(j * 16, 16)] for j in range(HH // 16)]

        off0 = pl.multiple_of(s * EPT, 8)
        pltpu.sync_copy(src_h.at[pl.ds(off0, CH)], src_c.at[0])
        pltpu.sync_copy(dst_h.at[pl.ds(off0, CH)], dst_c.at[0])
        pltpu.sync_copy(ea_h.at[pl.ds(off0, CH)], ea_c.at[0])
        off1 = pl.multiple_of(s * EPT + CH, 8)
        pltpu.sync_copy(src_h.at[pl.ds(off1, CH)], src_c.at[1])
        pltpu.sync_copy(dst_h.at[pl.ds(off1, CH)], dst_c.at[1])
        pltpu.sync_copy(ea_h.at[pl.ds(off1, CH)], ea_c.at[1])
        _idx_issue(2, 2)
        pltpu.async_copy(A_h.at[dst_c.at[0]], ab.at[0], sema).wait()
        pltpu.async_copy(B_h.at[src_c.at[0]], ab.at[0], semb,
                         add=True).wait()
        pltpu.async_copy(A_h.at[dst_c.at[1]], ab.at[1], sema)

        def _chunk(k, _):
            p3 = lax.rem(k, 3)
            s1 = lax.rem(k + 1, 3)
            s2 = lax.rem(k + 2, 3)
            j2 = lax.rem(k + 2, 4)
            j3 = lax.rem(k + 3, 4)
            kn1 = jnp.minimum(k + 1, NCHUNK - 1)
            kn2 = jnp.minimum(k + 2, NCHUNK - 1)
            kn3 = jnp.minimum(k + 3, NCHUNK - 1)
            _idx_wait(kn2, j2)

            @pl.when(k > 0)
            def _():
                pltpu.make_async_copy(ab.at[s2], ssh.at[dst_c.at[0]],
                                      sems).wait()

            pltpu.make_async_copy(A_h.at[dst_c.at[0]], ab.at[s1],
                                  sema).wait()
            pltpu.async_copy(B_h.at[src_c.at[lax.rem(kn1, 4)]], ab.at[s1],
                             semb, add=True)
            pltpu.async_copy(A_h.at[dst_c.at[j2]], ab.at[s2], sema)
            _idx_issue(kn3, j3)
            az_ = ab.at[p3]
            ek = ea_c.at[lax.rem(k, 4)]

            def _grp(g, _):
                ea16 = ek[pl.ds(g * 16, 16)]

                def _lane(h2, _):
                    for i in range(2):
                        l = h2 * 2 + i
                        r = g * 16 + l
                        eav = ea16[jnp.full((16,), l, jnp.int32)]
                        arow = az_.at[r]
                        for j in range(HH // 16):
                            aj = arow[pl.ds(j * 16, 16)]
                            arow[pl.ds(j * 16, 16)] = jnp.maximum(
                                aj + eav * w1[j], 0.0)
                    return 0

                lax.fori_loop(0, 8, _lane, 0)
                return 0

            lax.fori_loop(0, CH // 16, _grp, 0)
            pltpu.async_copy(az_, ssh.at[dst_c.at[lax.rem(k, 4)]], sems,
                             add=True)
            pltpu.make_async_copy(B_h.at[src_c.at[0]], ab.at[s1],
                                  semb).wait()
            return 0

        lax.fori_loop(0, NCHUNK, _chunk, 0)
        pltpu.make_async_copy(ab.at[0], ssh.at[dst_c.at[0]], sems).wait()
        pltpu.make_async_copy(A_h.at[dst_c.at[0]], ab.at[0], sema).wait()
        _idx_wait(0, 0)
        plsc.subcore_barrier()
        _write_ssh(S_h)

    @pl.when(c == 0)
    def _():
        _run_half(a0_h, b0_h, s0_h, 0)

    @pl.when(c == 1)
    def _():
        _run_half(a1_h, b1_h, s1_h, 1)

    if do_cnt:
        plsc.subcore_barrier()
        _fill(az, 0.0)
        _zero_ssh()
        _fill(oz, 1.0)
        plsc.subcore_barrier()
        half = NCHUNK // 2
        lo = c * half
        hi_ = jnp.where(c == 0, half, NCHUNK)

        def _cchunk(k, _):
            off = pl.multiple_of(s * EPT + k * CH, 8)
            dk = dst_c.at[0]
            pltpu.sync_copy(dst_h.at[pl.ds(off, CH)], dk)
            pltpu.sync_copy(oz, ssh.at[dk], add=True)
            return 0

        lax.fori_loop(lo, hi_, _cchunk, 0)
        plsc.subcore_barrier()

        @pl.when(c == 0)
        def _():
            _write_ssh(cnt_h)

        @pl.when(c == 1)
        def _():
            _write_ssh(cnt1_h)


def _make_sc_edge(do_cnt):
    out_type = [jax.ShapeDtypeStruct((N, HH), _f32),
                jax.ShapeDtypeStruct((N, HH), _f32)]
    if do_cnt:
        out_type = out_type + [jax.ShapeDtypeStruct((N, HH), _f32)] * 2
    scratch = [pltpu.VMEM((4, CH), jnp.int32),
               pltpu.VMEM((4, CH), jnp.int32),
               pltpu.VMEM((4, CH), _f32),
               pltpu.VMEM((3, CH, HH), _f32),
               pltpu.VMEM((1, HH), _f32)]
    scratch = scratch + [
               pltpu.VMEM_SHARED((N, HH), _f32),
               pltpu.SemaphoreType.DMA,
               pltpu.SemaphoreType.DMA,
               pltpu.SemaphoreType.DMA,
               pltpu.SemaphoreType.DMA]
    mesh = plsc.VectorSubcoreMesh(core_axis_name="c", subcore_axis_name="s")
    return functools.partial(
        pl.kernel, mesh=mesh, out_type=out_type, scratch_types=scratch,
    )(functools.partial(_sc_edge_body, do_cnt))


_SC_CACHE = {}


def _sc_edge_first(*args):
    if True not in _SC_CACHE:
        _SC_CACHE[True] = _make_sc_edge(True)
    return _SC_CACHE[True](*args)


def _sc_edge(*args):
    if False not in _SC_CACHE:
        _SC_CACHE[False] = _make_sc_edge(False)
    return _SC_CACHE[False](*args)


def _tc_pre_body(xp, nwp, nbr, cnd, cw1p, cb1r, cw2, cb2r, tp, tw1p, tb1r,
                 tw2, tb2r, w1a, w1b, b1r, h_out, a0, a1, b0, b1o):
    cond = jnp.dot(jax.nn.relu(
        jnp.dot(cnd[...], cw1p[...], preferred_element_type=_f32) + cb1r[...]),
        cw2[...], preferred_element_type=_f32) + cb2r[...]
    tf = jnp.dot(jax.nn.relu(
        jnp.dot(tp[...], tw1p[...], preferred_element_type=_f32) + tb1r[...]),
        tw2[...], preferred_element_type=_f32) + tb2r[...]
    h = (jnp.dot(xp[...], nwp[...], preferred_element_type=_f32)
         + nbr[...] + cond + tf)
    h_out[...] = h
    a = jnp.dot(h, w1a[...], preferred_element_type=_f32) + b1r[...]
    b = jnp.dot(h, w1b[...], preferred_element_type=_f32)
    a0[...] = a[:, :HH]
    a1[...] = a[:, HH:]
    b0[...] = b[:, :HH]
    b1o[...] = b[:, HH:]


def _agg_h(s0, s1, cnt, h, w2, b2r):
    cntv = cnt[...][:, :1]
    inv = 1.0 / jnp.maximum(cntv, 1.0)
    has = (cntv > 0.0).astype(_f32)
    w2v = w2[...]
    agg = (jnp.dot(s0[...], w2v[:HH, :], preferred_element_type=_f32)
           + jnp.dot(s1[...], w2v[HH:, :], preferred_element_type=_f32))
    agg = inv * agg + has * b2r[...]
    return jnp.maximum(agg, 0.0) + h[...]


def _tc_mid_body(s0, s1, cnt, h, w2, b2r, w1a, w1b, b1r,
                 h_out, a0, a1, b0, b1o):
    hn = _agg_h(s0, s1, cnt, h, w2, b2r)
    h_out[...] = hn
    a = jnp.dot(hn, w1a[...], preferred_element_type=_f32) + b1r[...]
    b = jnp.dot(hn, w1b[...], preferred_element_type=_f32)
    a0[...] = a[:, :HH]
    a1[...] = a[:, HH:]
    b0[...] = b[:, :HH]
    b1o[...] = b[:, HH:]


def _tc_last_body(s0, s1, cnt, h, w2, b2r, ow1, ob1r, ow2p, ob2r, out):
    hn = _agg_h(s0, s1, cnt, h, w2, b2r)
    o = jnp.dot(jax.nn.relu(
        jnp.dot(hn, ow1[...], preferred_element_type=_f32) + ob1r[...]),
        ow2p[...], preferred_element_type=_f32) + ob2r[...]
    out[...] = o


def _row_spec(w):
    return pl.BlockSpec((BS, w), lambda i: (i, 0))


def _full_spec(shape):
    nd = len(shape)
    return pl.BlockSpec(shape, lambda i: (0,) * nd)


_W = _full_spec((H, H))
_R = _full_spec((1, H))

_tc_pre = pl.pallas_call(
    _tc_pre_body,
    grid=(NBLK,),
    in_specs=[_row_spec(HH), _full_spec((HH, H)), _R,
              _full_spec((1, HH)), _full_spec((HH, H)), _R, _W, _R,
              _full_spec((1, HH)), _full_spec((HH, H)), _R, _W, _R,
              _W, _W, _R],
    out_specs=[_row_spec(H), _row_spec(HH), _row_spec(HH),
               _row_spec(HH), _row_spec(HH)],
    out_shape=[jax.ShapeDtypeStruct((N, H), _f32)] +
              [jax.ShapeDtypeStruct((N, HH), _f32)] * 4,
)

_tc_mid = pl.pallas_call(
    _tc_mid_body,
    grid=(NBLK,),
    in_specs=[_row_spec(HH), _row_spec(HH), _row_spec(HH), _row_spec(H),
              _W, _R, _W, _W, _R],
    out_specs=[_row_spec(H), _row_spec(HH), _row_spec(HH),
               _row_spec(HH), _row_spec(HH)],
    out_shape=[jax.ShapeDtypeStruct((N, H), _f32)] +
              [jax.ShapeDtypeStruct((N, HH), _f32)] * 4,
)

_tc_last = pl.pallas_call(
    _tc_last_body,
    grid=(NBLK,),
    in_specs=[_row_spec(HH), _row_spec(HH), _row_spec(HH), _row_spec(H),
              _W, _R, _W, _R, _full_spec((H, HH)), _full_spec((1, HH))],
    out_specs=_row_spec(HH),
    out_shape=jax.ShapeDtypeStruct((N, HH), _f32),
)


def kernel(x, edge_index, edge_attr, t, condition, cW1, cb1, cW2, cb2, nW, nb,
           tW1, tb1, tW2, tb2, convW1, convb1, convW2, convb2, oW1, ob1, oW2,
           ob2):
    src2 = edge_index[0]
    dst2 = edge_index[1]
    ea2 = edge_attr.reshape(E)

    xp = jnp.pad(x, ((0, 0), (0, HH - x.shape[1])))
    nwp = jnp.pad(nW, ((0, HH - nW.shape[0]), (0, 0)))
    cnd = jnp.pad(condition.reshape(1, -1), ((0, 0), (0, HH - condition.shape[0])))
    cw1p = jnp.pad(cW1, ((0, HH - cW1.shape[0]), (0, 0)))
    tp = jnp.pad(t.reshape(1, 1), ((0, 0), (0, HH - 1)))
    tw1p = jnp.pad(tW1, ((0, HH - tW1.shape[0]), (0, 0)))
    ow2p = jnp.pad(oW2, ((0, 0), (0, HH - oW2.shape[1])))
    ob2r = jnp.pad(ob2.reshape(1, -1), ((0, 0), (0, HH - ob2.shape[0])))
    w1e = convW1[:, 2 * H, :].reshape(L, 2, 1, HH)

    h, a0, a1, b0, b1 = _tc_pre(
        xp, nwp, nb.reshape(1, H), cnd, cw1p, cb1.reshape(1, H), cW2,
        cb2.reshape(1, H), tp, tw1p, tb1.reshape(1, H), tW2,
        tb2.reshape(1, H), convW1[0][:H], convW1[0][H:2 * H],
        convb1[0].reshape(1, H))

    s0, s1, cnt0, cnt1 = _sc_edge_first(a0, a1, b0, b1, src2, dst2, ea2,
                                        w1e[0])
    cnt = cnt0 + cnt1

    for i in range(1, L):
        h, a0, a1, b0, b1 = _tc_mid(
            s0, s1, cnt, h, convW2[i - 1], convb2[i - 1].reshape(1, H),
            convW1[i][:H], convW1[i][H:2 * H], convb1[i].reshape(1, H))
        s0, s1 = _sc_edge(a0, a1, b0, b1, src2, dst2, ea2, w1e[i])

    out = _tc_last(s0, s1, cnt, h, convW2[L - 1], convb2[L - 1].reshape(1, H),
                   oW1, ob1.reshape(1, H), ow2p, ob2r)
    return out[:, :x.shape[1]]
